# speedup vs baseline: 1.1592x; 1.1592x over previous
import functools

import jax
import jax.numpy as jnp
from jax import lax
from jax.experimental import pallas as pl
from jax.experimental.pallas import tpu as pltpu
from jax.experimental.pallas import tpu_sc as plsc

_POOL = 100000
_D = 128
_B = 4096
_S = 26
_TOTAL = _B * _S
_NW = 32
_PER_W = _TOTAL // _NW
_CHUNK = 128
_NCHUNK = _PER_W // _CHUNK


def _make_kernel():
    mesh = plsc.VectorSubcoreMesh(core_axis_name="c", subcore_axis_name="s")

    @functools.partial(
        pl.kernel,
        mesh=mesh,
        out_type=jax.ShapeDtypeStruct((_TOTAL, _D), jnp.float32),
        scratch_types=[
            pltpu.VMEM((_NCHUNK, _CHUNK), jnp.int32),
            pltpu.VMEM((_CHUNK, _D), jnp.float32),
            pltpu.SemaphoreType.DMA,
        ],
    )
    def gather_kernel(idx_hbm, table_hbm, out_hbm, idx_v, buf, sem):
        wid = lax.axis_index("s") * 2 + lax.axis_index("c")
        base = wid * _PER_W
        pltpu.sync_copy(idx_hbm.at[wid], idx_v)

        def body(j, carry):
            pltpu.async_copy(table_hbm.at[idx_v.at[j]], buf, sem).wait()
            pltpu.sync_copy(buf, out_hbm.at[pl.ds(base + j * _CHUNK, _CHUNK)])
            return carry

        lax.fori_loop(0, _NCHUNK, body, 0)

    return gather_kernel


_gather = _make_kernel()


def kernel(indices, embedding):
    idx = indices.astype(jnp.int32).reshape(_NW, _NCHUNK, _CHUNK)
    out = _gather(idx, embedding)
    return out.reshape(_B, _S, _D)


# speedup vs baseline: 1.2932x; 1.1156x over previous
import functools

import jax
import jax.numpy as jnp
from jax import lax
from jax.experimental import pallas as pl
from jax.experimental.pallas import tpu as pltpu
from jax.experimental.pallas import tpu_sc as plsc

_POOL = 100000
_D = 128
_B = 4096
_S = 26
_TOTAL = _B * _S
_NW = 32
_PER_W = _TOTAL // _NW
_CHUNK = 128
_NCHUNK = _PER_W // _CHUNK


_NBUF = 4
_PRIME = _NBUF - 1


def _make_kernel():
    mesh = plsc.VectorSubcoreMesh(core_axis_name="c", subcore_axis_name="s")

    @functools.partial(
        pl.kernel,
        mesh=mesh,
        out_type=jax.ShapeDtypeStruct((_TOTAL, _D), jnp.float32),
        scratch_types=(
            [pltpu.VMEM((_NCHUNK, _CHUNK), jnp.int32),
             pltpu.VMEM((_NBUF, _CHUNK, _D), jnp.float32)]
            + [pltpu.SemaphoreType.DMA] * (2 * _NBUF)
        ),
    )
    def gather_kernel(idx_hbm, table_hbm, out_hbm, idx_v, bufs, *sems):
        gsems = sems[:_NBUF]
        wsems = sems[_NBUF:]
        wid = lax.axis_index("s") * 2 + lax.axis_index("c")
        base = wid * _PER_W
        pltpu.sync_copy(idx_hbm.at[wid], idx_v)

        def out_rows(j):
            return out_hbm.at[pl.ds(base + j * _CHUNK, _CHUNK)]

        for j in range(_PRIME):
            pltpu.async_copy(table_hbm.at[idx_v.at[j]], bufs.at[j], gsems[j])

        for j in range(_NCHUNK):
            b = j % _NBUF
            pltpu.make_async_copy(table_hbm.at[idx_v.at[j]], bufs.at[b], gsems[b]).wait()
            pltpu.async_copy(bufs.at[b], out_rows(j), wsems[b])
            nj = j + _PRIME
            if nj < _NCHUNK:
                nb = nj % _NBUF
                if nj >= _NBUF:
                    pj = nj - _NBUF
                    pltpu.make_async_copy(bufs.at[nb], out_rows(pj), wsems[nb]).wait()
                pltpu.async_copy(table_hbm.at[idx_v.at[nj]], bufs.at[nb], gsems[nb])

        for j in range(_NCHUNK - _NBUF, _NCHUNK):
            b = j % _NBUF
            pltpu.make_async_copy(bufs.at[b], out_rows(j), wsems[b]).wait()

    return gather_kernel


_gather = _make_kernel()


def kernel(indices, embedding):
    idx = indices.astype(jnp.int32).reshape(_NW, _NCHUNK, _CHUNK)
    out = _gather(idx, embedding)
    return out.reshape(_B, _S, _D)


# speedup vs baseline: 1.9191x; 1.4839x over previous
import functools

import jax
import jax.numpy as jnp
from jax import lax
from jax.experimental import pallas as pl
from jax.experimental.pallas import tpu as pltpu
from jax.experimental.pallas import tpu_sc as plsc

_POOL = 100000
_D = 128
_B = 4096
_S = 26
_NW = 32
_BPW = _B // _NW
_SUB = 4
_NCHUNK = _BPW // _SUB
_NBUF = 3
_PRIME = _NBUF - 1


def _make_kernel():
    mesh = plsc.VectorSubcoreMesh(core_axis_name="c", subcore_axis_name="s")

    @functools.partial(
        pl.kernel,
        mesh=mesh,
        out_type=jax.ShapeDtypeStruct((_B, _S, _D), jnp.float32),
        scratch_types=(
            [pltpu.VMEM((_BPW, _S), jnp.int32),
             pltpu.VMEM((_NBUF, _SUB, _S, _D), jnp.float32)]
            + [pltpu.SemaphoreType.DMA] * (2 * _NBUF)
        ),
    )
    def gather_kernel(idx_hbm, table_hbm, out_hbm, idx_v, bufs, *sems):
        gsems = sems[:_NBUF]
        wsems = sems[_NBUF:]
        wid = lax.axis_index("s") * 2 + lax.axis_index("c")
        wb = wid * _BPW
        pltpu.sync_copy(idx_hbm.at[pl.ds(wb, _BPW)], idx_v)

        def gathers(c, b, issue):
            for k in range(_SUB):
                dsc = pltpu.make_async_copy(
                    table_hbm.at[idx_v.at[c * _SUB + k]], bufs.at[b, k], gsems[b])
                if issue:
                    dsc.start()
                else:
                    dsc.wait()

        def wb_copy(c, b):
            return pltpu.make_async_copy(
                bufs.at[b], out_hbm.at[pl.ds(wb + c * _SUB, _SUB)], wsems[b])

        for c in range(_PRIME):
            gathers(c, c, issue=True)

        for c in range(_NCHUNK):
            b = c % _NBUF
            gathers(c, b, issue=False)
            wb_copy(c, b).start()
            nc = c + _PRIME
            if nc < _NCHUNK:
                nb = nc % _NBUF
                if nc >= _NBUF:
                    wb_copy(nc - _NBUF, nb).wait()
                gathers(nc, nb, issue=True)

        for c in range(_NCHUNK - _NBUF, _NCHUNK):
            wb_copy(c, c % _NBUF).wait()

    return gather_kernel


_gather = _make_kernel()


def kernel(indices, embedding):
    return _gather(indices.astype(jnp.int32), embedding)


# speedup vs baseline: 3.6898x; 1.9227x over previous
import functools

import jax
import jax.numpy as jnp
from jax import lax
from jax.experimental import pallas as pl
from jax.experimental.pallas import tpu as pltpu
from jax.experimental.pallas import tpu_sc as plsc

_POOL = 100000
_D = 128
_B = 4096
_S = 26
_NW = 32
_BPW = _B // _NW
_NBUF = 4
_PRIME = _NBUF - 1


def _make_kernel():
    mesh = plsc.VectorSubcoreMesh(core_axis_name="c", subcore_axis_name="s")

    @functools.partial(
        pl.kernel,
        mesh=mesh,
        out_type=jax.ShapeDtypeStruct((_S, _B, _D), jnp.float32),
        scratch_types=(
            [pltpu.VMEM((_S, _BPW), jnp.int32),
             pltpu.VMEM((_NBUF, _BPW, _D), jnp.float32)]
            + [pltpu.SemaphoreType.DMA] * (2 * _NBUF)
        ),
    )
    def gather_kernel(idx_hbm, table_hbm, out_hbm, idx_v, bufs, *sems):
        gsems = sems[:_NBUF]
        wsems = sems[_NBUF:]
        wid = lax.axis_index("s") * 2 + lax.axis_index("c")
        wb = wid * _BPW
        pltpu.sync_copy(idx_hbm.at[wid], idx_v)

        def g_copy(s, b):
            return pltpu.make_async_copy(
                table_hbm.at[idx_v.at[s]], bufs.at[b], gsems[b])

        def w_copy(s, b):
            return pltpu.make_async_copy(
                bufs.at[b], out_hbm.at[s, pl.ds(wb, _BPW)], wsems[b])

        for s in range(_PRIME):
            g_copy(s, s).start()

        for s in range(_S):
            b = s % _NBUF
            g_copy(s, b).wait()
            w_copy(s, b).start()
            ns = s + _PRIME
            if ns < _S:
                nb = ns % _NBUF
                if ns >= _NBUF:
                    w_copy(ns - _NBUF, nb).wait()
                g_copy(ns, nb).start()

        for s in range(_S - _NBUF, _S):
            w_copy(s, s % _NBUF).wait()

    return gather_kernel


_gather = _make_kernel()


def kernel(indices, embedding):
    idx = indices.astype(jnp.int32).T.reshape(_S, _NW, _BPW).transpose(1, 0, 2)
    out = _gather(idx, embedding)
    return out.transpose(1, 0, 2)


# speedup vs baseline: 3.7629x; 1.0198x over previous
import functools

import jax
import jax.numpy as jnp
from jax import lax
from jax.experimental import pallas as pl
from jax.experimental.pallas import tpu as pltpu
from jax.experimental.pallas import tpu_sc as plsc

_POOL = 100000
_D = 128
_B = 4096
_S = 26
_NW = 32
_BPW = _B // _NW
_NBUF = 6
_PRIME = _NBUF - 1


def _make_kernel():
    mesh = plsc.VectorSubcoreMesh(core_axis_name="c", subcore_axis_name="s")

    @functools.partial(
        pl.kernel,
        mesh=mesh,
        out_type=jax.ShapeDtypeStruct((_S, _B, _D), jnp.float32),
        scratch_types=(
            [pltpu.VMEM((_S, _BPW), jnp.int32),
             pltpu.VMEM((_NBUF, _BPW, _D), jnp.float32)]
            + [pltpu.SemaphoreType.DMA] * (2 * _NBUF)
        ),
    )
    def gather_kernel(idx_hbm, table_hbm, out_hbm, idx_v, bufs, *sems):
        gsems = sems[:_NBUF]
        wsems = sems[_NBUF:]
        wid = lax.axis_index("s") * 2 + lax.axis_index("c")
        wb = wid * _BPW
        pltpu.sync_copy(idx_hbm.at[wid], idx_v)

        def g_copy(s, b):
            return pltpu.make_async_copy(
                table_hbm.at[idx_v.at[s]], bufs.at[b], gsems[b])

        def w_copy(s, b):
            return pltpu.make_async_copy(
                bufs.at[b], out_hbm.at[s, pl.ds(wb, _BPW)], wsems[b])

        for s in range(_PRIME):
            g_copy(s, s).start()

        for s in range(_S):
            b = s % _NBUF
            g_copy(s, b).wait()
            w_copy(s, b).start()
            ns = s + _PRIME
            if ns < _S:
                nb = ns % _NBUF
                if ns >= _NBUF:
                    w_copy(ns - _NBUF, nb).wait()
                g_copy(ns, nb).start()

        for s in range(_S - _NBUF, _S):
            w_copy(s, s % _NBUF).wait()

    return gather_kernel


_gather = _make_kernel()


def kernel(indices, embedding):
    idx = indices.astype(jnp.int32).T.reshape(_S, _NW, _BPW).transpose(1, 0, 2)
    out = _gather(idx, embedding)
    return out.transpose(1, 0, 2)
